# baseline (speedup 1.0000x reference)
import functools

import jax
import jax.numpy as jnp
from jax import lax
from jax.experimental import pallas as pl
from jax.experimental.pallas import tpu as pltpu
from jax.experimental.pallas import tpu_sc as plsc

_B, _N, _D, _L = 1024, 1000, 128, 200
_NC, _NS = 2, 16
_NW = _NC * _NS
_BPW = _B // _NW
_PR = 336
_POFF = (0, 336, 664)
_PLO = (0, 336, 672)
_PHI = (336, 672, 1000)
_DV = _D // 16
_NCH = _L // 16


def _sc_body(table, mi1, mi2, out,
             blk, i1v, i2v, orow, semb, semi, semo):
    wid = lax.axis_index("s") * _NC + lax.axis_index("c")
    base = wid * _BPW

    lane = lax.iota(jnp.int32, 16)

    def fire_idx(bi, p):
        pltpu.async_copy(mi1.at[base + bi], i1v.at[p], semi)
        pltpu.async_copy(mi2.at[base + bi], i2v.at[p], semi)

    def drain_idx(p):
        pltpu.make_async_copy(mi1.at[base], i1v.at[p], semi).wait()
        pltpu.make_async_copy(mi2.at[base], i2v.at[p], semi).wait()

    def fire_blk(bi, t, b):
        row0 = (base + bi) * _N + _POFF[t]
        pltpu.async_copy(table.at[pl.ds(row0, _PR)],
                         blk.at[b, pl.ds(0, _PR)], semb)

    def wait_blk():
        pltpu.make_async_copy(table.at[pl.ds(0, _PR)],
                              blk.at[0, pl.ds(0, _PR)], semb).wait()

    def part_pass(t, b, p, acc1, acc2, with_cnt):
        off = _POFF[t]
        lo = _PLO[t]
        hi = _PHI[t]

        def do_rows(mich, jlo, acc):
            ich = jnp.bitwise_and(mich, 1023)
            mch = lax.shift_right_logical(mich, 10)
            sel = (mch * jnp.where(ich >= lo, 1, 0)
                   * jnp.where(ich < hi, 1, 0))
            lidx = jnp.where(sel > 0, ich - off, _PR)
            for j in range(jlo, 16):
                row = lidx[j]
                acc = tuple(acc[k] + blk[b, row, pl.ds(16 * k, 16)]
                            for k in range(_DV))
            return acc

        def chunk_body(c, carry):
            acc1 = carry[:_DV]
            acc2 = carry[_DV:2 * _DV]
            mich1 = i1v[p, pl.ds(c * 16, 16)]
            mich2 = i2v[p, pl.ds(c * 16, 16)]
            acc1 = do_rows(mich1, 0, acc1)
            acc2 = do_rows(mich2, 0, acc2)
            if with_cnt:
                cnt1 = carry[2 * _DV] + lax.shift_right_logical(mich1, 10)
                cnt2 = carry[2 * _DV + 1] + lax.shift_right_logical(mich2, 10)
                return acc1 + acc2 + (cnt1, cnt2)
            return acc1 + acc2

        carry0 = acc1 + acc2
        if with_cnt:
            carry0 = carry0 + (jnp.zeros((16,), jnp.int32),
                               jnp.zeros((16,), jnp.int32))
        carry = lax.fori_loop(0, _NCH, chunk_body, carry0)
        acc1 = carry[:_DV]
        acc2 = carry[_DV:2 * _DV]
        him = jnp.where(lane >= 8, 1, 0)
        michm1 = i1v[p, pl.ds(_L - 16, 16)] * him
        michm2 = i2v[p, pl.ds(_L - 16, 16)] * him
        acc1 = do_rows(michm1, 8, acc1)
        acc2 = do_rows(michm2, 8, acc2)
        cnt1 = cnt2 = None
        if with_cnt:
            cnt1 = carry[2 * _DV] + lax.shift_right_logical(michm1, 10)
            cnt2 = carry[2 * _DV + 1] + lax.shift_right_logical(michm2, 10)
        return acc1, acc2, cnt1, cnt2

    def inv_of(cnt):
        tot = cnt[0]
        for j in range(1, 16):
            tot = tot + cnt[j]
        totv = jnp.zeros((16,), jnp.int32) + tot
        return 1.0 / jnp.maximum(totv.astype(jnp.float32), 1.0)

    zacc = tuple(jnp.zeros((16,), jnp.float32) for _ in range(_DV))

    zf = jnp.zeros((16,), jnp.float32)
    for bb in range(2):
        for k in range(_DV):
            blk[bb, _PR, pl.ds(16 * k, 16)] = zf

    fire_idx(0, 0)
    fire_blk(0, 0, 0)

    def per_batch(bi, carry):
        p = lax.rem(bi, 2)
        q = p
        r = 1 - p
        drain_idx(p)

        wait_blk()
        fire_blk(bi, 1, r)
        a1, a2, cnt1, cnt2 = part_pass(0, q, p, zacc, zacc, True)

        wait_blk()
        fire_blk(bi, 2, q)
        a1, a2, _u1, _u2 = part_pass(1, r, p, a1, a2, False)

        wait_blk()

        @pl.when(bi + 1 < _BPW)
        def _():
            fire_blk(bi + 1, 0, r)
            fire_idx(bi + 1, 1 - p)

        a1, a2, _u3, _u4 = part_pass(2, q, p, a1, a2, False)

        inv1 = inv_of(cnt1)
        inv2 = inv_of(cnt2)

        @pl.when(bi >= 2)
        def _():
            pltpu.make_async_copy(orow.at[p], out.at[base], semo).wait()

        for k in range(_DV):
            orow[p, pl.ds(16 * k, 16)] = a1[k] * inv1 + a2[k] * inv2
        pltpu.async_copy(orow.at[p], out.at[base + bi], semo)
        return carry

    lax.fori_loop(0, _BPW, per_batch, 0)
    pltpu.make_async_copy(orow.at[0], out.at[base], semo).wait()
    pltpu.make_async_copy(orow.at[1], out.at[base], semo).wait()


_node_model_sc = functools.partial(
    pl.kernel,
    out_type=jax.ShapeDtypeStruct((_B, _D), jnp.float32),
    mesh=plsc.VectorSubcoreMesh(core_axis_name="c", subcore_axis_name="s"),
    scratch_types=[
        pltpu.VMEM((2, _PR + 1, _D), jnp.float32),
        pltpu.VMEM((2, _L), jnp.int32),
        pltpu.VMEM((2, _L), jnp.int32),
        pltpu.VMEM((2, _D), jnp.float32),
        pltpu.SemaphoreType.DMA,
        pltpu.SemaphoreType.DMA,
        pltpu.SemaphoreType.DMA,
    ],
)(_sc_body)


def kernel(graph_embed, graph_event1, graph_event1_mask,
           graph_event2, graph_event2_mask):
    table = graph_embed.reshape(_B * _N, _D)
    mi1 = jnp.bitwise_or(
        graph_event1.astype(jnp.int32),
        lax.shift_left(jnp.where(graph_event1_mask != 0, 1, 0)
                       .astype(jnp.int32), 10))
    mi2 = jnp.bitwise_or(
        graph_event2.astype(jnp.int32),
        lax.shift_left(jnp.where(graph_event2_mask != 0, 1, 0)
                       .astype(jnp.int32), 10))
    return _node_model_sc(table, mi1, mi2)


# speedup vs baseline: 1.1299x; 1.1299x over previous
import functools

import jax
import jax.numpy as jnp
from jax import lax
from jax.experimental import pallas as pl
from jax.experimental.pallas import tpu as pltpu
from jax.experimental.pallas import tpu_sc as plsc

_B, _N, _D, _L = 1024, 1000, 128, 200
_NC, _NS = 2, 16
_NW = _NC * _NS
_BPW = _B // _NW
_PR = 336
_POFF = (0, 336, 664)
_PLO = (0, 336, 672)
_PHI = (336, 672, 1000)
_DV = _D // 16
_NCH = _L // 16


def _sc_body(table, mi1, mi2, out,
             blk, i1v, i2v, orow, semb, semi, semo):
    wid = lax.axis_index("s") * _NC + lax.axis_index("c")
    base = wid * _BPW

    lane = lax.iota(jnp.int32, 16)

    def fire_idx(bi, p):
        pltpu.async_copy(mi1.at[base + bi], i1v.at[p], semi)
        pltpu.async_copy(mi2.at[base + bi], i2v.at[p], semi)

    def drain_idx(p):
        pltpu.make_async_copy(mi1.at[base], i1v.at[p], semi).wait()
        pltpu.make_async_copy(mi2.at[base], i2v.at[p], semi).wait()

    def fire_blk(bi, t, b):
        row0 = (base + bi) * _N + _POFF[t]
        pltpu.async_copy(table.at[pl.ds(row0, _PR)],
                         blk.at[b, pl.ds(0, _PR)], semb)

    def wait_blk():
        pltpu.make_async_copy(table.at[pl.ds(0, _PR)],
                              blk.at[0, pl.ds(0, _PR)], semb).wait()

    def part_pass(t, b, p, mi_ref, acc, with_cnt):
        off = _POFF[t]
        lo = _PLO[t]
        hi = _PHI[t]

        def do_rows(mich, jlo, acc):
            ich = jnp.bitwise_and(mich, 1023)
            mch = lax.shift_right_logical(mich, 10)
            sel = (mch * jnp.where(ich >= lo, 1, 0)
                   * jnp.where(ich < hi, 1, 0))
            lidx = jnp.where(sel > 0, ich - off, _PR)
            for j in range(jlo, 16):
                row = lidx[j]
                acc = tuple(acc[k] + blk[b, row, pl.ds(16 * k, 16)]
                            for k in range(_DV))
            return acc

        def chunk_body(c, carry):
            acc = carry[:_DV]
            mich = mi_ref[p, pl.ds(c * 16, 16)]
            acc = do_rows(mich, 0, acc)
            if with_cnt:
                cnt = carry[_DV] + lax.shift_right_logical(mich, 10)
                return acc + (cnt,)
            return acc

        carry0 = acc
        if with_cnt:
            carry0 = carry0 + (jnp.zeros((16,), jnp.int32),)
        carry = lax.fori_loop(0, _NCH, chunk_body, carry0)
        acc = carry[:_DV]
        mich = mi_ref[p, pl.ds(_L - 16, 16)]
        michm = mich * jnp.where(lane >= 8, 1, 0)
        acc = do_rows(michm, 8, acc)
        cnt = None
        if with_cnt:
            cnt = carry[_DV] + lax.shift_right_logical(michm, 10)
        return acc, cnt

    def inv_of(cnt):
        tot = cnt[0]
        for j in range(1, 16):
            tot = tot + cnt[j]
        totv = jnp.zeros((16,), jnp.int32) + tot
        return 1.0 / jnp.maximum(totv.astype(jnp.float32), 1.0)

    zacc = tuple(jnp.zeros((16,), jnp.float32) for _ in range(_DV))

    zf = jnp.zeros((16,), jnp.float32)
    for bb in range(2):
        for k in range(_DV):
            blk[bb, _PR, pl.ds(16 * k, 16)] = zf

    fire_idx(0, 0)
    fire_blk(0, 0, 0)

    def per_batch(bi, carry):
        p = lax.rem(bi, 2)
        q = p
        r = 1 - p
        drain_idx(p)

        wait_blk()
        fire_blk(bi, 1, r)
        a1, cnt1 = part_pass(0, q, p, i1v, zacc, True)
        a2, cnt2 = part_pass(0, q, p, i2v, zacc, True)

        wait_blk()
        fire_blk(bi, 2, q)
        a1, _u1 = part_pass(1, r, p, i1v, a1, False)
        a2, _u2 = part_pass(1, r, p, i2v, a2, False)

        wait_blk()

        @pl.when(bi + 1 < _BPW)
        def _():
            fire_blk(bi + 1, 0, r)
            fire_idx(bi + 1, 1 - p)

        a1, _u3 = part_pass(2, q, p, i1v, a1, False)
        a2, _u4 = part_pass(2, q, p, i2v, a2, False)

        inv1 = inv_of(cnt1)
        inv2 = inv_of(cnt2)

        @pl.when(bi >= 2)
        def _():
            pltpu.make_async_copy(orow.at[p], out.at[base], semo).wait()

        for k in range(_DV):
            orow[p, pl.ds(16 * k, 16)] = a1[k] * inv1 + a2[k] * inv2
        pltpu.async_copy(orow.at[p], out.at[base + bi], semo)
        return carry

    lax.fori_loop(0, _BPW, per_batch, 0)
    pltpu.make_async_copy(orow.at[0], out.at[base], semo).wait()
    pltpu.make_async_copy(orow.at[1], out.at[base], semo).wait()


_node_model_sc = functools.partial(
    pl.kernel,
    out_type=jax.ShapeDtypeStruct((_B, _D), jnp.float32),
    mesh=plsc.VectorSubcoreMesh(core_axis_name="c", subcore_axis_name="s"),
    scratch_types=[
        pltpu.VMEM((2, _PR + 1, _D), jnp.float32),
        pltpu.VMEM((2, _L), jnp.int32),
        pltpu.VMEM((2, _L), jnp.int32),
        pltpu.VMEM((2, _D), jnp.float32),
        pltpu.SemaphoreType.DMA,
        pltpu.SemaphoreType.DMA,
        pltpu.SemaphoreType.DMA,
    ],
)(_sc_body)


def kernel(graph_embed, graph_event1, graph_event1_mask,
           graph_event2, graph_event2_mask):
    table = graph_embed.reshape(_B * _N, _D)
    mi1 = jnp.bitwise_or(
        graph_event1.astype(jnp.int32),
        lax.shift_left(jnp.where(graph_event1_mask != 0, 1, 0)
                       .astype(jnp.int32), 10))
    mi2 = jnp.bitwise_or(
        graph_event2.astype(jnp.int32),
        lax.shift_left(jnp.where(graph_event2_mask != 0, 1, 0)
                       .astype(jnp.int32), 10))
    return _node_model_sc(table, mi1, mi2)
